# baseline (speedup 1.0000x reference)
import functools

import jax
import jax.numpy as jnp
from jax import lax
from jax.experimental import pallas as pl
from jax.experimental.pallas import tpu as pltpu
from jax.experimental.pallas import tpu_sc as plsc
from jax.experimental.layout import Layout, with_layout_constraint

BATCH = 16384
EMBED_K = 32
NROWS = 1000000
NC = 2
NS = 16
NW = NC * NS
BPW = BATCH // NW
OCT = NROWS // 8
KW = 8
NWAVE = EMBED_K // KW
FLAT = EMBED_K * NROWS


def _mf_body(uidx_hbm, iidx_hbm, ut_pairs, it_pairs,
             dot_hbm, uemb_hbm, iemb_hbm,
             uidx_v, iidx_v, uq_v, up_v, iq_v, ip_v,
             upairs, ipairs, urows, irows, dots_v, sem):
    wid = lax.axis_index("s") * NC + lax.axis_index("c")
    base = wid * BPW

    pltpu.sync_copy(uidx_hbm.at[wid], uidx_v)
    pltpu.sync_copy(iidx_hbm.at[wid], iidx_v)

    def qcomp(g, carry):
        sl = pl.ds(g * 16, 16)
        uv = uidx_v[sl]
        iv = iidx_v[sl]
        uq_v[sl] = uv >> 3
        up_v[sl] = uv & 7
        iq_v[sl] = iv >> 3
        ip_v[sl] = iv & 7
        return carry

    lax.fori_loop(0, BPW // 16, qcomp, 0)

    for w in range(NWAVE):
        copies = []
        for j in range(KW):
            k = w * KW + j
            copies.append(pltpu.async_copy(
                ut_pairs.at[pl.ds(k * OCT, OCT)].at[uq_v],
                upairs.at[pl.ds(j * BPW, BPW)], sem))
            copies.append(pltpu.async_copy(
                it_pairs.at[pl.ds(k * OCT, OCT)].at[iq_v],
                ipairs.at[pl.ds(j * BPW, BPW)], sem))
        for c in copies:
            c.wait()

        def ext_grp(g, carry, w=w):
            sl = pl.ds(g * 16, 16)
            b16 = g * 16 + lax.iota(jnp.int32, 16)
            pu = up_v[sl]
            pi = ip_v[sl]
            acc = jnp.zeros((16,), jnp.float32) if w == 0 else dots_v[sl]
            for j in range(KW):
                kk = jnp.full((16,), w * KW + j, jnp.int32)
                uv = plsc.load_gather(upairs, [j * BPW + b16, pu])
                iv = plsc.load_gather(ipairs, [j * BPW + b16, pi])
                plsc.store_scatter(urows, [b16, kk], uv)
                plsc.store_scatter(irows, [b16, kk], iv)
                acc = acc + uv * iv
            dots_v[sl] = acc
            return carry

        lax.fori_loop(0, BPW // 16, ext_grp, 0)

    pltpu.sync_copy(urows, uemb_hbm.at[pl.ds(base, BPW)])
    pltpu.sync_copy(irows, iemb_hbm.at[pl.ds(base, BPW)])
    pltpu.sync_copy(dots_v, dot_hbm.at[pl.ds(base, BPW)])


@functools.partial(jax.jit, static_argnames=())
def _mf(uidx, iidx, ut_pairs, it_pairs):
    kern = pl.kernel(
        _mf_body,
        out_type=[
            jax.ShapeDtypeStruct((BATCH,), jnp.float32),
            jax.ShapeDtypeStruct((BATCH, EMBED_K), jnp.float32),
            jax.ShapeDtypeStruct((BATCH, EMBED_K), jnp.float32),
        ],
        mesh=plsc.VectorSubcoreMesh(core_axis_name="c", subcore_axis_name="s"),
        scratch_types=[
            pltpu.VMEM((BPW,), jnp.int32),
            pltpu.VMEM((BPW,), jnp.int32),
            pltpu.VMEM((BPW,), jnp.int32),
            pltpu.VMEM((BPW,), jnp.int32),
            pltpu.VMEM((BPW,), jnp.int32),
            pltpu.VMEM((BPW,), jnp.int32),
            pltpu.VMEM((KW * BPW, 8), jnp.float32),
            pltpu.VMEM((KW * BPW, 8), jnp.float32),
            pltpu.VMEM((BPW, EMBED_K), jnp.float32),
            pltpu.VMEM((BPW, EMBED_K), jnp.float32),
            pltpu.VMEM((BPW,), jnp.float32),
            pltpu.SemaphoreType.DMA,
        ],
        compiler_params=pltpu.CompilerParams(
            needs_layout_passes=False, use_tc_tiling_on_sc=False),
    )
    return kern(uidx, iidx, ut_pairs, it_pairs)


def kernel(x, user_table, item_table):
    xi = x.astype(jnp.int32)
    uidx = xi[:, 0].reshape(NW, BPW)
    iidx = xi[:, 1].reshape(NW, BPW)
    row_major = Layout((0, 1))
    ut_lin = with_layout_constraint(user_table.T, row_major)
    it_lin = with_layout_constraint(item_table.T, row_major)
    ut_pairs = ut_lin.reshape(FLAT // 8, 8)
    it_pairs = it_lin.reshape(FLAT // 8, 8)
    dots, uemb, iemb = _mf(uidx, iidx, ut_pairs, it_pairs)
    return (dots[:, None], uemb, iemb)


# speedup vs baseline: 5.5872x; 5.5872x over previous
import functools

import jax
import jax.numpy as jnp
from jax import lax
from jax.experimental import pallas as pl
from jax.experimental.pallas import tpu as pltpu
from jax.experimental.pallas import tpu_sc as plsc

BATCH = 16384
EMBED_K = 32
NC = 2
NS = 16
NW = NC * NS
BPW = BATCH // NW
IDX_MINOR = 128
NGRP = BPW // IDX_MINOR


def _mf_body(uidx_hbm, iidx_hbm, user_table, item_table,
             out_hbm, uemb_hbm, iemb_hbm,
             uidx_v, iidx_v, urows_v, irows_v, out_v, sem):
    wid = lax.axis_index("s") * NC + lax.axis_index("c")
    base = wid * BPW

    pltpu.sync_copy(uidx_hbm.at[wid], uidx_v)
    pltpu.sync_copy(iidx_hbm.at[wid], iidx_v)

    copies = []
    for j in range(NGRP):
        copies.append(pltpu.async_copy(
            user_table.at[uidx_v.at[j]],
            urows_v.at[pl.ds(j * IDX_MINOR, IDX_MINOR)], sem))
        copies.append(pltpu.async_copy(
            item_table.at[iidx_v.at[j]],
            irows_v.at[pl.ds(j * IDX_MINOR, IDX_MINOR)], sem))
    for c in copies:
        c.wait()

    last_lane = lax.iota(jnp.int32, 16) == 15

    def dot_row(i, carry):
        u0 = urows_v[i, pl.ds(0, 16)]
        u1 = urows_v[i, pl.ds(16, 16)]
        v0 = irows_v[i, pl.ds(0, 16)]
        v1 = irows_v[i, pl.ds(16, 16)]
        w = u0 * v0 + u1 * v1
        s = plsc.cumsum(w)
        plsc.store_scatter(out_v, [jnp.full((16,), i, jnp.int32)], s,
                           mask=last_lane)
        return carry

    lax.fori_loop(0, BPW, dot_row, 0, unroll=8)

    pltpu.sync_copy(urows_v, uemb_hbm.at[pl.ds(base, BPW)])
    pltpu.sync_copy(irows_v, iemb_hbm.at[pl.ds(base, BPW)])
    pltpu.sync_copy(out_v, out_hbm.at[pl.ds(base, BPW)])


@functools.partial(jax.jit, static_argnames=())
def _mf(uidx, iidx, user_table, item_table):
    kern = pl.kernel(
        _mf_body,
        out_type=[
            jax.ShapeDtypeStruct((BATCH,), jnp.float32),
            jax.ShapeDtypeStruct((BATCH, EMBED_K), jnp.float32),
            jax.ShapeDtypeStruct((BATCH, EMBED_K), jnp.float32),
        ],
        mesh=plsc.VectorSubcoreMesh(core_axis_name="c", subcore_axis_name="s"),
        scratch_types=[
            pltpu.VMEM((NGRP, IDX_MINOR), jnp.int32),
            pltpu.VMEM((NGRP, IDX_MINOR), jnp.int32),
            pltpu.VMEM((BPW, EMBED_K), jnp.float32),
            pltpu.VMEM((BPW, EMBED_K), jnp.float32),
            pltpu.VMEM((BPW,), jnp.float32),
            pltpu.SemaphoreType.DMA,
        ],
        compiler_params=pltpu.CompilerParams(
            needs_layout_passes=False, use_tc_tiling_on_sc=False),
    )
    return kern(uidx, iidx, user_table, item_table)


def kernel(x, user_table, item_table):
    xi = x.astype(jnp.int32)
    uidx = xi[:, 0].reshape(NW, NGRP, IDX_MINOR)
    iidx = xi[:, 1].reshape(NW, NGRP, IDX_MINOR)
    out, uemb, iemb = _mf(uidx, iidx, user_table, item_table)
    return (out[:, None], uemb, iemb)
